# baseline (speedup 1.0000x reference)
import functools

import jax
import jax.numpy as jnp
from jax.experimental import pallas as pl
from jax.experimental.pallas import tpu as pltpu


def _pick_block(n):
    for r in (1280, 640, 320, 160, 80, 40, 16, 8):
        if n % r == 0:
            return r
    return n


def _fused_kernel(meta_ref, x_ref, w_enc_ref, b_enc_ref,
                  w1_ref, b1_ref, gamma1_ref, beta1_ref, w2_ref, b2_ref,
                  out_ref, acc_ref, *, nb, r, s_total):
    i = pl.program_id(0)

    @pl.when(i == 0)
    def _init():
        acc_ref[...] = jnp.full_like(acc_ref, -jnp.inf)

    emb = jnp.dot(x_ref[...], w_enc_ref[...],
                  preferred_element_type=jnp.float32)

    base = i * r
    rowid = jax.lax.broadcasted_iota(jnp.int32, (r, 1), 0)

    def _seg_of(row):
        def bs(_, lohi):
            lo, hi = lohi
            mid = (lo + hi) // 2
            go = meta_ref[mid] <= row
            return jnp.where(go, mid, lo), jnp.where(go, hi, mid)
        lo, _ = jax.lax.fori_loop(
            0, 10, bs, (jnp.int32(0), jnp.int32(s_total + 1)))
        return lo

    b_first = _seg_of(base)
    b_last = _seg_of(base + r - 1)

    def body(k, _):
        s = b_first + k
        a = meta_ref[s] - base
        e = meta_ref[s + 1] - base
        mask = (rowid >= a) & (rowid < e)
        red = jnp.max(jnp.where(mask, emb, -jnp.inf),
                      axis=0, keepdims=True)
        acc_ref[pl.ds(s, 1), :] = jnp.maximum(acc_ref[pl.ds(s, 1), :], red)
        return 0

    jax.lax.fori_loop(0, b_last - b_first + 1, body, 0)

    @pl.when(i == nb - 1)
    def _final():
        g = acc_ref[...] + b_enc_ref[...]
        h = jnp.dot(g, w1_ref[...],
                    preferred_element_type=jnp.float32) + b1_ref[...]
        mu = jnp.mean(h, axis=-1, keepdims=True)
        var = jnp.mean((h - mu) * (h - mu), axis=-1, keepdims=True)
        h = (h - mu) * jax.lax.rsqrt(var + 1e-5) * gamma1_ref[...] \
            + beta1_ref[...]
        h = jnp.maximum(h, 0.0)
        out_ref[...] = jnp.dot(h, w2_ref[...],
                               preferred_element_type=jnp.float32) \
            + b2_ref[...]


def kernel(x, batch, W_enc, b_enc, W1, b1, gamma1, beta1, W2, b2):
    n, d = x.shape
    h = W1.shape[1]
    nt = W2.shape[1]
    s_total = 512
    r = _pick_block(n)
    nb = n // r

    batch = batch.astype(jnp.int32)
    meta = jnp.searchsorted(batch,
                            jnp.arange(s_total + 1, dtype=jnp.int32),
                            side="left").astype(jnp.int32)

    grid_spec = pltpu.PrefetchScalarGridSpec(
        num_scalar_prefetch=1,
        grid=(nb,),
        in_specs=[
            pl.BlockSpec((r, d), lambda i, meta: (i, 0)),
            pl.BlockSpec((d, d), lambda i, meta: (0, 0)),
            pl.BlockSpec((1, d), lambda i, meta: (0, 0)),
            pl.BlockSpec((d, h), lambda i, meta: (0, 0)),
            pl.BlockSpec((1, h), lambda i, meta: (0, 0)),
            pl.BlockSpec((1, h), lambda i, meta: (0, 0)),
            pl.BlockSpec((1, h), lambda i, meta: (0, 0)),
            pl.BlockSpec((h, nt), lambda i, meta: (0, 0)),
            pl.BlockSpec((1, nt), lambda i, meta: (0, 0)),
        ],
        out_specs=pl.BlockSpec((s_total, nt), lambda i, meta: (0, 0)),
        scratch_shapes=[pltpu.VMEM((s_total, d), jnp.float32)],
    )

    fn = functools.partial(_fused_kernel, nb=nb, r=r, s_total=s_total)
    out = pl.pallas_call(
        fn,
        grid_spec=grid_spec,
        out_shape=jax.ShapeDtypeStruct((s_total, nt), jnp.float32),
    )(meta, x, W_enc, b_enc.reshape(1, d),
      W1, b1.reshape(1, h), gamma1.reshape(1, h), beta1.reshape(1, h),
      W2, b2.reshape(1, nt))
    return out


# speedup vs baseline: 1.6953x; 1.6953x over previous
import functools

import jax
import jax.numpy as jnp
from jax.experimental import pallas as pl
from jax.experimental.pallas import tpu as pltpu


def _pick_block(n):
    for r in (1280, 640, 320, 160, 80, 40, 16, 8):
        if n % r == 0:
            return r
    return n


def _fused_kernel(meta_ref, x_ref, w_enc_ref, b_enc_ref,
                  w1_ref, b1_ref, gamma1_ref, beta1_ref, w2_ref, b2_ref,
                  out_ref, acc_ref, *, nb, r, s_total):
    i = pl.program_id(0)

    @pl.when(i == 0)
    def _init():
        acc_ref[...] = jnp.full_like(acc_ref, -jnp.inf)

    emb = jnp.dot(x_ref[...], w_enc_ref[...],
                  preferred_element_type=jnp.float32)

    b_first = meta_ref[2 * i]
    b_last = meta_ref[2 * i + 1]
    base = i * r
    rowid = jax.lax.broadcasted_iota(jnp.int32, (r, 1), 0)

    def body(k, _):
        s = b_first + k
        a = meta_ref[2 * nb + s] - base
        e = meta_ref[2 * nb + s + 1] - base
        mask = (rowid >= a) & (rowid < e)
        red = jnp.max(jnp.where(mask, emb, -jnp.inf),
                      axis=0, keepdims=True)
        acc_ref[pl.ds(s, 1), :] = jnp.maximum(acc_ref[pl.ds(s, 1), :], red)
        return 0

    jax.lax.fori_loop(0, b_last - b_first + 1, body, 0)

    @pl.when(i == nb - 1)
    def _final():
        g = acc_ref[...] + b_enc_ref[...]
        h = jnp.dot(g, w1_ref[...],
                    preferred_element_type=jnp.float32) + b1_ref[...]
        mu = jnp.mean(h, axis=-1, keepdims=True)
        var = jnp.mean((h - mu) * (h - mu), axis=-1, keepdims=True)
        h = (h - mu) * jax.lax.rsqrt(var + 1e-5) * gamma1_ref[...] \
            + beta1_ref[...]
        h = jnp.maximum(h, 0.0)
        out_ref[...] = jnp.dot(h, w2_ref[...],
                               preferred_element_type=jnp.float32) \
            + b2_ref[...]


def kernel(x, batch, W_enc, b_enc, W1, b1, gamma1, beta1, W2, b2):
    n, d = x.shape
    h = W1.shape[1]
    nt = W2.shape[1]
    s_total = 512
    r = _pick_block(n)
    nb = n // r

    batch = batch.astype(jnp.int32)
    blk = batch.reshape(nb, r)
    per_block = jnp.stack([blk[:, 0], blk[:, -1]], axis=1).reshape(-1)
    off = jnp.searchsorted(batch, jnp.arange(s_total + 1, dtype=jnp.int32),
                           side="left").astype(jnp.int32)
    meta = jnp.concatenate([per_block, off])

    grid_spec = pltpu.PrefetchScalarGridSpec(
        num_scalar_prefetch=1,
        grid=(nb,),
        in_specs=[
            pl.BlockSpec((r, d), lambda i, meta: (i, 0)),
            pl.BlockSpec((d, d), lambda i, meta: (0, 0)),
            pl.BlockSpec((1, d), lambda i, meta: (0, 0)),
            pl.BlockSpec((d, h), lambda i, meta: (0, 0)),
            pl.BlockSpec((1, h), lambda i, meta: (0, 0)),
            pl.BlockSpec((1, h), lambda i, meta: (0, 0)),
            pl.BlockSpec((1, h), lambda i, meta: (0, 0)),
            pl.BlockSpec((h, nt), lambda i, meta: (0, 0)),
            pl.BlockSpec((1, nt), lambda i, meta: (0, 0)),
        ],
        out_specs=pl.BlockSpec((s_total, nt), lambda i, meta: (0, 0)),
        scratch_shapes=[pltpu.VMEM((s_total, d), jnp.float32)],
    )

    fn = functools.partial(_fused_kernel, nb=nb, r=r, s_total=s_total)
    out = pl.pallas_call(
        fn,
        grid_spec=grid_spec,
        out_shape=jax.ShapeDtypeStruct((s_total, nt), jnp.float32),
    )(meta, x, W_enc, b_enc.reshape(1, d),
      W1, b1.reshape(1, h), gamma1.reshape(1, h), beta1.reshape(1, h),
      W2, b2.reshape(1, nt))
    return out


# speedup vs baseline: 1.8069x; 1.0658x over previous
import functools

import jax
import jax.numpy as jnp
from jax.experimental import pallas as pl
from jax.experimental.pallas import tpu as pltpu


def _pick_block(n):
    for r in (1280, 640, 320, 160, 80, 40, 16, 8):
        if n % r == 0:
            return r
    return n


def _fused_kernel(meta_ref, xa_ref, xb_ref, w_enc_ref, b_enc_ref,
                  w1_ref, b1_ref, gamma1_ref, beta1_ref, w2_ref, b2_ref,
                  out_ref, acc_ref, *, nb, r, s_total):
    i = pl.program_id(0)
    rh = r // 2

    @pl.when(i == 0)
    def _init():
        acc_ref[...] = jnp.full_like(acc_ref, -jnp.inf)

    w_enc = w_enc_ref[...]
    emb_a = jnp.dot(xa_ref[...], w_enc, preferred_element_type=jnp.float32)
    emb_b = jnp.dot(xb_ref[...], w_enc, preferred_element_type=jnp.float32)

    b_first = meta_ref[2 * i]
    b_last = meta_ref[2 * i + 1]
    base = i * r
    rowid = jax.lax.broadcasted_iota(jnp.int32, (rh, 1), 0)

    def body(k, _):
        s = b_first + k
        a = meta_ref[2 * nb + s] - base
        e = meta_ref[2 * nb + s + 1] - base
        mask_a = (rowid >= a) & (rowid < e)
        mask_b = (rowid >= a - rh) & (rowid < e - rh)
        red = jnp.maximum(
            jnp.max(jnp.where(mask_a, emb_a, -jnp.inf),
                    axis=0, keepdims=True),
            jnp.max(jnp.where(mask_b, emb_b, -jnp.inf),
                    axis=0, keepdims=True))
        acc_ref[pl.ds(s, 1), :] = jnp.maximum(acc_ref[pl.ds(s, 1), :], red)
        return 0

    jax.lax.fori_loop(0, b_last - b_first + 1, body, 0)

    @pl.when(i == nb - 1)
    def _final():
        g = acc_ref[...] + b_enc_ref[...]
        h = jnp.dot(g, w1_ref[...],
                    preferred_element_type=jnp.float32) + b1_ref[...]
        mu = jnp.mean(h, axis=-1, keepdims=True)
        var = jnp.mean((h - mu) * (h - mu), axis=-1, keepdims=True)
        h = (h - mu) * jax.lax.rsqrt(var + 1e-5) * gamma1_ref[...] \
            + beta1_ref[...]
        h = jnp.maximum(h, 0.0)
        out_ref[...] = jnp.dot(h, w2_ref[...],
                               preferred_element_type=jnp.float32) \
            + b2_ref[...]


def kernel(x, batch, W_enc, b_enc, W1, b1, gamma1, beta1, W2, b2):
    n, d = x.shape
    h = W1.shape[1]
    nt = W2.shape[1]
    s_total = 512
    r = _pick_block(n)
    nb = n // r

    batch = batch.astype(jnp.int32)
    blk = batch.reshape(nb, r)
    per_block = jnp.stack([blk[:, 0], blk[:, -1]], axis=1).reshape(-1)
    off = jnp.searchsorted(batch, jnp.arange(s_total + 1, dtype=jnp.int32),
                           side="left").astype(jnp.int32)
    meta = jnp.concatenate([per_block, off])

    grid_spec = pltpu.PrefetchScalarGridSpec(
        num_scalar_prefetch=1,
        grid=(nb,),
        in_specs=[
            pl.BlockSpec((r // 2, d), lambda i, meta: (2 * i, 0)),
            pl.BlockSpec((r // 2, d), lambda i, meta: (2 * i + 1, 0)),
            pl.BlockSpec((d, d), lambda i, meta: (0, 0)),
            pl.BlockSpec((1, d), lambda i, meta: (0, 0)),
            pl.BlockSpec((d, h), lambda i, meta: (0, 0)),
            pl.BlockSpec((1, h), lambda i, meta: (0, 0)),
            pl.BlockSpec((1, h), lambda i, meta: (0, 0)),
            pl.BlockSpec((1, h), lambda i, meta: (0, 0)),
            pl.BlockSpec((h, nt), lambda i, meta: (0, 0)),
            pl.BlockSpec((1, nt), lambda i, meta: (0, 0)),
        ],
        out_specs=pl.BlockSpec((s_total, nt), lambda i, meta: (0, 0)),
        scratch_shapes=[pltpu.VMEM((s_total, d), jnp.float32)],
    )

    fn = functools.partial(_fused_kernel, nb=nb, r=r, s_total=s_total)
    out = pl.pallas_call(
        fn,
        grid_spec=grid_spec,
        out_shape=jax.ShapeDtypeStruct((s_total, nt), jnp.float32),
    )(meta, x, x, W_enc, b_enc.reshape(1, d),
      W1, b1.reshape(1, h), gamma1.reshape(1, h), beta1.reshape(1, h),
      W2, b2.reshape(1, nt))
    return out


# speedup vs baseline: 1.9180x; 1.0615x over previous
import functools

import jax
import jax.numpy as jnp
from jax.experimental import pallas as pl
from jax.experimental.pallas import tpu as pltpu


def _pick_block(n):
    for r in (1280, 640, 320, 160, 80, 40, 16, 8):
        if n % r == 0:
            return r
    return n


def _fused_kernel(meta_ref, *refs, nb, r, s_total, ns):
    (x_refs, (w_enc_ref, b_enc_ref, w1_ref, b1_ref, gamma1_ref,
              beta1_ref, w2_ref, b2_ref, out_ref, acc_ref)) = \
        refs[:ns], refs[ns:]
    i = pl.program_id(0)
    rh = r // ns

    @pl.when(i == 0)
    def _init():
        acc_ref[...] = jnp.full_like(acc_ref, -jnp.inf)

    w_enc = w_enc_ref[...]
    embs = [jnp.dot(xr[...], w_enc, preferred_element_type=jnp.float32)
            for xr in x_refs]

    b_first = meta_ref[2 * i]
    b_last = meta_ref[2 * i + 1]
    base = i * r
    rowid = jax.lax.broadcasted_iota(jnp.int32, (rh, 1), 0)

    def body(k, _):
        s = b_first + k
        a = meta_ref[2 * nb + s] - base
        e = meta_ref[2 * nb + s + 1] - base
        red = jnp.full((1, embs[0].shape[-1]), -jnp.inf, jnp.float32)
        for j, emb in enumerate(embs):
            mask = (rowid >= a - j * rh) & (rowid < e - j * rh)
            red = jnp.maximum(
                red, jnp.max(jnp.where(mask, emb, -jnp.inf),
                             axis=0, keepdims=True))
        acc_ref[pl.ds(s, 1), :] = jnp.maximum(acc_ref[pl.ds(s, 1), :], red)
        return 0

    jax.lax.fori_loop(0, b_last - b_first + 1, body, 0)

    @pl.when(i == nb - 1)
    def _final():
        g = acc_ref[...] + b_enc_ref[...]
        h = jnp.dot(g, w1_ref[...],
                    preferred_element_type=jnp.float32) + b1_ref[...]
        mu = jnp.mean(h, axis=-1, keepdims=True)
        var = jnp.mean((h - mu) * (h - mu), axis=-1, keepdims=True)
        h = (h - mu) * jax.lax.rsqrt(var + 1e-5) * gamma1_ref[...] \
            + beta1_ref[...]
        h = jnp.maximum(h, 0.0)
        out_ref[...] = jnp.dot(h, w2_ref[...],
                               preferred_element_type=jnp.float32) \
            + b2_ref[...]


def kernel(x, batch, W_enc, b_enc, W1, b1, gamma1, beta1, W2, b2):
    n, d = x.shape
    h = W1.shape[1]
    nt = W2.shape[1]
    s_total = 512
    r = _pick_block(n)
    nb = n // r
    ns = 4

    batch = batch.astype(jnp.int32)
    blk = batch.reshape(nb, r)
    per_block = jnp.stack([blk[:, 0], blk[:, -1]], axis=1).reshape(-1)
    off = jnp.searchsorted(batch, jnp.arange(s_total + 1, dtype=jnp.int32),
                           side="left").astype(jnp.int32)
    meta = jnp.concatenate([per_block, off])

    grid_spec = pltpu.PrefetchScalarGridSpec(
        num_scalar_prefetch=1,
        grid=(nb,),
        in_specs=[
            pl.BlockSpec((r // ns, d),
                         functools.partial(
                             lambda j, i, meta: (ns * i + j, 0), j))
            for j in range(ns)
        ] + [
            pl.BlockSpec((d, d), lambda i, meta: (0, 0)),
            pl.BlockSpec((1, d), lambda i, meta: (0, 0)),
            pl.BlockSpec((d, h), lambda i, meta: (0, 0)),
            pl.BlockSpec((1, h), lambda i, meta: (0, 0)),
            pl.BlockSpec((1, h), lambda i, meta: (0, 0)),
            pl.BlockSpec((1, h), lambda i, meta: (0, 0)),
            pl.BlockSpec((h, nt), lambda i, meta: (0, 0)),
            pl.BlockSpec((1, nt), lambda i, meta: (0, 0)),
        ],
        out_specs=pl.BlockSpec((s_total, nt), lambda i, meta: (0, 0)),
        scratch_shapes=[pltpu.VMEM((s_total, d), jnp.float32)],
    )

    fn = functools.partial(_fused_kernel, nb=nb, r=r, s_total=s_total,
                           ns=ns)
    out = pl.pallas_call(
        fn,
        grid_spec=grid_spec,
        out_shape=jax.ShapeDtypeStruct((s_total, nt), jnp.float32),
    )(meta, *([x] * ns), W_enc, b_enc.reshape(1, d),
      W1, b1.reshape(1, h), gamma1.reshape(1, h), beta1.reshape(1, h),
      W2, b2.reshape(1, nt))
    return out
